# Initial kernel scaffold; baseline (speedup 1.0000x reference)
import jax
import jax.numpy as jnp
from jax.experimental import pallas as pl


def kernel(log_probs, targets):
    raise NotImplementedError("write your pallas kernel here")


# speedup vs baseline: 2.2807x; 2.2807x over previous
import functools
import math

import jax
import jax.numpy as jnp
from jax import lax
from jax.experimental import pallas as pl
from jax.experimental.pallas import tpu as pltpu
from jax.experimental.pallas import tpu_sc as plsc

_PAD = 1
_SMOOTH = 0.1

_info = plsc.get_sparse_core_info()
_NC, _NS, _L = _info.num_cores, _info.num_subcores, _info.num_lanes
_NW = _NC * _NS


def _make_sc_gather(N, V):
    rpw = N // _NW
    nslice = rpw // _L
    mesh = plsc.VectorSubcoreMesh(core_axis_name="c", subcore_axis_name="s")

    @functools.partial(
        pl.kernel,
        mesh=mesh,
        out_type=jax.ShapeDtypeStruct((_NW, 128), jnp.float32),
        scratch_types=[
            pltpu.VMEM((rpw,), jnp.int32),
            pltpu.VMEM((2 * rpw,), jnp.int32),
            pltpu.VMEM((2 * rpw,), jnp.float32),
            pltpu.VMEM((128,), jnp.float32),
            pltpu.SemaphoreType.DMA,
        ],
    )
    def k(lp_hbm, t_hbm, out_hbm, t_v, idx_v, val_v, res_v, sem):
        wid = lax.axis_index("s") * _NC + lax.axis_index("c")
        base = wid * rpw
        pltpu.sync_copy(t_hbm.at[pl.ds(base, rpw)], t_v)
        lane = lax.broadcasted_iota(jnp.int32, (_L,), 0)
        for s in range(nslice):
            t16 = t_v[pl.ds(s * _L, _L)]
            rows = (base + s * _L) * V + lane * V
            idx_v[pl.ds(s * _L, _L)] = rows + t16
            idx_v[pl.ds(rpw + s * _L, _L)] = rows + _PAD
        pltpu.async_copy(lp_hbm.at[idx_v], val_v, sem).wait()
        zero = jnp.zeros((_L,), jnp.float32)
        one = jnp.ones((_L,), jnp.float32)
        acc_t = zero
        acc_p = zero
        acc_c = zero
        for s in range(nslice):
            m = t_v[pl.ds(s * _L, _L)] != _PAD
            acc_t = acc_t + jnp.where(m, val_v[pl.ds(s * _L, _L)], zero)
            acc_p = acc_p + jnp.where(m, val_v[pl.ds(rpw + s * _L, _L)], zero)
            acc_c = acc_c + jnp.where(m, one, zero)
        res_v[pl.ds(0, _L)] = acc_t
        res_v[pl.ds(_L, _L)] = acc_p
        res_v[pl.ds(2 * _L, _L)] = acc_c
        for s in range(3, 128 // _L):
            res_v[pl.ds(s * _L, _L)] = zero
        pltpu.sync_copy(res_v, out_hbm.at[wid])

    return k


def _make_tc_sum(N, V, rb, cb, eps, centropy):
    nr, nc = N // rb, V // cb

    def body(t_ref, lp_ref, g_ref, out_ref, acc_ref):
        i = pl.program_id(0)
        j = pl.program_id(1)
        first = jnp.logical_and(i == 0, j == 0)
        last = jnp.logical_and(i == nr - 1, j == nc - 1)
        t = t_ref[0, 0, :].reshape(rb, 1)
        part = jnp.sum(jnp.where(t != _PAD, lp_ref[...], 0.0))

        @pl.when(first)
        def _():
            acc_ref[0] = part

        @pl.when(jnp.logical_not(first))
        def _():
            acc_ref[0] += part

        @pl.when(last)
        def _():
            lane = lax.broadcasted_iota(jnp.int32, (_NW, 128), 1)
            coef = jnp.where(
                lane < _L,
                eps - (1.0 - _SMOOTH),
                jnp.where(lane < 2 * _L, eps,
                          jnp.where(lane < 3 * _L, centropy, 0.0)),
            )
            out_ref[0, 0] = jnp.sum(g_ref[...] * coef) - eps * acc_ref[0]

    return pl.pallas_call(
        body,
        grid=(nr, nc),
        in_specs=[
            pl.BlockSpec((1, 1, rb), lambda i, j: (i, 0, 0)),
            pl.BlockSpec((rb, cb), lambda i, j: (i, j)),
            pl.BlockSpec((_NW, 128), lambda i, j: (0, 0)),
        ],
        out_specs=pl.BlockSpec(memory_space=pltpu.SMEM),
        out_shape=jax.ShapeDtypeStruct((1, 1), jnp.float32),
        scratch_shapes=[pltpu.SMEM((1,), jnp.float32)],
    )


def kernel(log_probs, targets):
    b, s, v = log_probs.shape
    n = b * s
    eps = _SMOOTH / (v - 2)
    centropy = (1.0 - _SMOOTH) * math.log(1.0 - _SMOOTH) + _SMOOTH * math.log(eps)

    t_flat = targets.reshape(n).astype(jnp.int32)
    partials = _make_sc_gather(n, v)(log_probs.reshape(n * v), t_flat)

    rb, cb = 256, 6400
    t3 = t_flat.reshape(n // rb, 1, rb)
    out = _make_tc_sum(n, v, rb, cb, eps, centropy)(
        t3, log_probs.reshape(n, v), partials
    )
    return out[0, 0]


# speedup vs baseline: 2.3968x; 1.0509x over previous
import functools
import math

import jax
import jax.numpy as jnp
from jax import lax
from jax.experimental import pallas as pl
from jax.experimental.pallas import tpu as pltpu
from jax.experimental.pallas import tpu_sc as plsc

_PAD = 1
_SMOOTH = 0.1

_info = plsc.get_sparse_core_info()
_NC, _NS, _L = _info.num_cores, _info.num_subcores, _info.num_lanes
_NW = _NC * _NS


def _make_sc_gather(N, V):
    rpw = N // _NW
    nslice = rpw // _L
    mesh = plsc.VectorSubcoreMesh(core_axis_name="c", subcore_axis_name="s")

    @functools.partial(
        pl.kernel,
        mesh=mesh,
        out_type=jax.ShapeDtypeStruct((_NW, 128), jnp.float32),
        scratch_types=[
            pltpu.VMEM((rpw,), jnp.int32),
            pltpu.VMEM((2 * rpw,), jnp.int32),
            pltpu.VMEM((2 * rpw,), jnp.float32),
            pltpu.VMEM((128,), jnp.float32),
            pltpu.SemaphoreType.DMA,
        ],
    )
    def k(lp_hbm, t_hbm, out_hbm, t_v, idx_v, val_v, res_v, sem):
        wid = lax.axis_index("s") * _NC + lax.axis_index("c")
        base = wid * rpw
        pltpu.sync_copy(t_hbm.at[pl.ds(base, rpw)], t_v)
        lane = lax.broadcasted_iota(jnp.int32, (_L,), 0)
        for s in range(nslice):
            t16 = t_v[pl.ds(s * _L, _L)]
            rows = (base + s * _L) * V + lane * V
            idx_v[pl.ds(s * _L, _L)] = rows + t16
            idx_v[pl.ds(rpw + s * _L, _L)] = rows + _PAD
        pltpu.async_copy(lp_hbm.at[idx_v], val_v, sem).wait()
        zero = jnp.zeros((_L,), jnp.float32)
        one = jnp.ones((_L,), jnp.float32)
        acc_t = zero
        acc_p = zero
        acc_c = zero
        for s in range(nslice):
            m = t_v[pl.ds(s * _L, _L)] != _PAD
            acc_t = acc_t + jnp.where(m, val_v[pl.ds(s * _L, _L)], zero)
            acc_p = acc_p + jnp.where(m, val_v[pl.ds(rpw + s * _L, _L)], zero)
            acc_c = acc_c + jnp.where(m, one, zero)
        res_v[pl.ds(0, _L)] = acc_t
        res_v[pl.ds(_L, _L)] = acc_p
        res_v[pl.ds(2 * _L, _L)] = acc_c
        for s in range(3, 128 // _L):
            res_v[pl.ds(s * _L, _L)] = zero
        pltpu.sync_copy(res_v, out_hbm.at[wid])

    return k


def _make_tc_sum(N, V, rb, cb, eps, centropy):
    nr, nc = N // rb, V // cb

    def body(t_ref, lp_ref, g_ref, out_ref, acc_ref):
        i = pl.program_id(0)
        j = pl.program_id(1)
        first = jnp.logical_and(i == 0, j == 0)
        last = jnp.logical_and(i == nr - 1, j == nc - 1)
        t = t_ref[0, 0, :].reshape(rb, 1)
        part = jnp.sum(jnp.where(t != _PAD, lp_ref[...], 0.0))

        @pl.when(first)
        def _():
            acc_ref[0] = part

        @pl.when(jnp.logical_not(first))
        def _():
            acc_ref[0] += part

        @pl.when(last)
        def _():
            lane = lax.broadcasted_iota(jnp.int32, (_NW, 128), 1)
            coef = jnp.where(
                lane < _L,
                eps - (1.0 - _SMOOTH),
                jnp.where(lane < 2 * _L, eps,
                          jnp.where(lane < 3 * _L, centropy, 0.0)),
            )
            out_ref[0, 0] = jnp.sum(g_ref[...] * coef) - eps * acc_ref[0]

    return pl.pallas_call(
        body,
        grid=(nr, nc),
        in_specs=[
            pl.BlockSpec((1, 1, rb), lambda i, j: (i, 0, 0)),
            pl.BlockSpec((rb, cb), lambda i, j: (i, j)),
            pl.BlockSpec((_NW, 128), lambda i, j: (0, 0)),
        ],
        out_specs=pl.BlockSpec(memory_space=pltpu.SMEM),
        out_shape=jax.ShapeDtypeStruct((1, 1), jnp.float32),
        scratch_shapes=[pltpu.SMEM((1,), jnp.float32)],
    )


def kernel(log_probs, targets):
    b, s, v = log_probs.shape
    n = b * s
    eps = _SMOOTH / (v - 2)
    centropy = (1.0 - _SMOOTH) * math.log(1.0 - _SMOOTH) + _SMOOTH * math.log(eps)

    t_flat = targets.reshape(n).astype(jnp.int32)
    partials = _make_sc_gather(n, v)(log_probs.reshape(n * v), t_flat)

    rb, cb = 128, 32000
    t3 = t_flat.reshape(n // rb, 1, rb)
    out = _make_tc_sum(n, v, rb, cb, eps, centropy)(
        t3, log_probs.reshape(n, v), partials
    )
    return out[0, 0]


# speedup vs baseline: 2.4337x; 1.0154x over previous
import functools
import math

import jax
import jax.numpy as jnp
from jax import lax
from jax.experimental import pallas as pl
from jax.experimental.pallas import tpu as pltpu
from jax.experimental.pallas import tpu_sc as plsc

_PAD = 1
_SMOOTH = 0.1

_info = plsc.get_sparse_core_info()
_NC, _NS, _L = _info.num_cores, _info.num_subcores, _info.num_lanes
_NW = _NC * _NS


def _make_sc_gather(N, V):
    rpw = N // _NW
    nslice = rpw // _L
    mesh = plsc.VectorSubcoreMesh(core_axis_name="c", subcore_axis_name="s")

    @functools.partial(
        pl.kernel,
        mesh=mesh,
        out_type=jax.ShapeDtypeStruct((_NW, 128), jnp.float32),
        scratch_types=[
            pltpu.VMEM((rpw,), jnp.int32),
            pltpu.VMEM((2 * rpw,), jnp.int32),
            pltpu.VMEM((2 * rpw,), jnp.float32),
            pltpu.VMEM((128,), jnp.float32),
            pltpu.SemaphoreType.DMA,
        ],
    )
    def k(lp_hbm, t_hbm, out_hbm, t_v, idx_v, val_v, res_v, sem):
        wid = lax.axis_index("s") * _NC + lax.axis_index("c")
        base = wid * rpw
        pltpu.sync_copy(t_hbm.at[pl.ds(base, rpw)], t_v)
        lane = lax.broadcasted_iota(jnp.int32, (_L,), 0)
        for s in range(nslice):
            t16 = t_v[pl.ds(s * _L, _L)]
            rows = (base + s * _L) * V + lane * V
            idx_v[pl.ds(s * _L, _L)] = rows + t16
            idx_v[pl.ds(rpw + s * _L, _L)] = rows + _PAD
        pltpu.async_copy(lp_hbm.at[idx_v], val_v, sem).wait()
        zero = jnp.zeros((_L,), jnp.float32)
        one = jnp.ones((_L,), jnp.float32)
        acc_t = zero
        acc_p = zero
        acc_c = zero
        for s in range(nslice):
            m = t_v[pl.ds(s * _L, _L)] != _PAD
            acc_t = acc_t + jnp.where(m, val_v[pl.ds(s * _L, _L)], zero)
            acc_p = acc_p + jnp.where(m, val_v[pl.ds(rpw + s * _L, _L)], zero)
            acc_c = acc_c + jnp.where(m, one, zero)
        res_v[pl.ds(0, _L)] = acc_t
        res_v[pl.ds(_L, _L)] = acc_p
        res_v[pl.ds(2 * _L, _L)] = acc_c
        for s in range(3, 128 // _L):
            res_v[pl.ds(s * _L, _L)] = zero
        pltpu.sync_copy(res_v, out_hbm.at[wid])

    return k


def _make_tc_sum(N, V, rb, eps, centropy):
    half = N // 2
    nsteps = half // rb

    def body(t_ref, lp0_ref, lp1_ref, g_ref, out_ref, acc_ref):
        i = pl.program_id(0)
        first = i == 0
        last = i == nsteps - 1
        t0 = t_ref[0, 0, :rb].reshape(rb, 1)
        t1 = t_ref[0, 0, rb:].reshape(rb, 1)
        part = jnp.sum(jnp.where(t0 != _PAD, lp0_ref[0], 0.0)) + jnp.sum(
            jnp.where(t1 != _PAD, lp1_ref[0], 0.0)
        )

        @pl.when(first)
        def _():
            acc_ref[0] = part

        @pl.when(jnp.logical_not(first))
        def _():
            acc_ref[0] += part

        @pl.when(last)
        def _():
            lane = lax.broadcasted_iota(jnp.int32, (_NW, 128), 1)
            coef = jnp.where(
                lane < _L,
                eps - (1.0 - _SMOOTH),
                jnp.where(lane < 2 * _L, eps,
                          jnp.where(lane < 3 * _L, centropy, 0.0)),
            )
            out_ref[0, 0] = jnp.sum(g_ref[...] * coef) - eps * acc_ref[0]

    return pl.pallas_call(
        body,
        grid=(nsteps,),
        in_specs=[
            pl.BlockSpec((1, 1, 2 * rb), lambda i: (i, 0, 0)),
            pl.BlockSpec((1, rb, V), lambda i: (0, i, 0)),
            pl.BlockSpec((1, rb, V), lambda i: (1, i, 0)),
            pl.BlockSpec((_NW, 128), lambda i: (0, 0)),
        ],
        out_specs=pl.BlockSpec(memory_space=pltpu.SMEM),
        out_shape=jax.ShapeDtypeStruct((1, 1), jnp.float32),
        scratch_shapes=[pltpu.SMEM((1,), jnp.float32)],
    )


def kernel(log_probs, targets):
    b, s, v = log_probs.shape
    n = b * s
    eps = _SMOOTH / (v - 2)
    centropy = (1.0 - _SMOOTH) * math.log(1.0 - _SMOOTH) + _SMOOTH * math.log(eps)

    t_flat = targets.reshape(n).astype(jnp.int32)
    partials = _make_sc_gather(n, v)(log_probs.reshape(n * v), t_flat)

    rb = 64
    nsteps = (n // 2) // rb
    t3 = (
        t_flat.reshape(2, nsteps, rb)
        .transpose(1, 0, 2)
        .reshape(nsteps, 1, 2 * rb)
    )
    lp3 = log_probs.reshape(2, n // 2, v)
    out = _make_tc_sum(n, v, rb, eps, centropy)(t3, lp3, lp3, partials)
    return out[0, 0]
